# baseline (speedup 1.0000x reference)
import functools

import jax
import jax.numpy as jnp
from jax import lax
from jax.experimental import pallas as pl
from jax.experimental.pallas import tpu as pltpu
from jax.experimental.pallas import tpu_sc as plsc

N, D, S, B, H, O = 100000, 128, 16, 2048, 128, 64
NC, NS = 2, 16
NW = NC * NS
BPW = B // NW
CHUNK = 16
NCHUNK = BPW // CHUNK
ROWS = CHUNK * S
LANES = 16


def _sc_gather_mean(nodes, node_features, neigh_idx):
    mesh = plsc.VectorSubcoreMesh(core_axis_name="c", subcore_axis_name="s")

    @functools.partial(
        pl.kernel,
        out_type=(
            jax.ShapeDtypeStruct((B, D), jnp.float32),
            jax.ShapeDtypeStruct((B, D), jnp.float32),
        ),
        mesh=mesh,
        compiler_params=pltpu.CompilerParams(use_tc_tiling_on_sc=False),
        scratch_types=[
            pltpu.VMEM((BPW,), jnp.int32),
            pltpu.VMEM((BPW, S), jnp.int32),
            pltpu.VMEM((BPW * S,), jnp.int32),
            pltpu.VMEM((BPW, D), jnp.float32),
            pltpu.VMEM((ROWS, D), jnp.float32),
            pltpu.VMEM((CHUNK, D), jnp.float32),
            pltpu.SemaphoreType.DMA,
        ],
    )
    def k(nodes_hbm, feats_hbm, nidx_hbm, orig_out, agg_out,
          seeds_v, nrows_v, flat_v, orig_v, gbuf, aggbuf, sem):
        wid = lax.axis_index("s") * NC + lax.axis_index("c")
        base = wid * BPW

        pltpu.sync_copy(nodes_hbm.at[pl.ds(base, BPW)], seeds_v)
        pltpu.async_copy(nidx_hbm.at[seeds_v], nrows_v, sem).wait()
        pltpu.async_copy(feats_hbm.at[seeds_v], orig_v, sem).wait()
        pltpu.sync_copy(orig_v, orig_out.at[pl.ds(base, BPW)])

        def flatten_body(i, carry):
            flat_v[pl.ds(i * S, S)] = nrows_v[i, :]
            return carry

        lax.fori_loop(0, BPW, flatten_body, 0)

        inv = jnp.float32(1.0 / S)
        for c in range(NCHUNK):
            pltpu.async_copy(
                feats_hbm.at[flat_v.at[pl.ds(c * ROWS, ROWS)]], gbuf, sem
            ).wait()

            def seed_body(s, carry):
                r0 = s * S
                for kk in range(D // LANES):
                    col = pl.ds(kk * LANES, LANES)
                    acc = gbuf[r0, col]
                    for j in range(1, S):
                        acc = acc + gbuf[r0 + j, col]
                    aggbuf[s, col] = acc * inv
                return carry

            lax.fori_loop(0, CHUNK, seed_body, 0)
            pltpu.sync_copy(aggbuf, agg_out.at[pl.ds(base + c * CHUNK, CHUNK)])

    return k(nodes, node_features, neigh_idx)


def _tc_dense(orig, agg, W2, Wout):
    W2aT = W2[:, :D].T
    W2bT = W2[:, D:].T
    WoutT = Wout.T
    BM = 256

    def body(o_ref, a_ref, w2a_ref, w2b_ref, wout_ref, out_ref):
        h = jnp.dot(o_ref[...], w2a_ref[...], preferred_element_type=jnp.float32)
        h = h + jnp.dot(a_ref[...], w2b_ref[...], preferred_element_type=jnp.float32)
        h = jnp.maximum(h, 0.0)
        logits = jnp.dot(h, wout_ref[...], preferred_element_type=jnp.float32)
        m = jnp.max(logits, axis=-1, keepdims=True)
        e = jnp.exp(logits - m)
        out_ref[...] = e / jnp.sum(e, axis=-1, keepdims=True)

    return pl.pallas_call(
        body,
        grid=(B // BM,),
        in_specs=[
            pl.BlockSpec((BM, D), lambda i: (i, 0)),
            pl.BlockSpec((BM, D), lambda i: (i, 0)),
            pl.BlockSpec((D, H), lambda i: (0, 0)),
            pl.BlockSpec((D, H), lambda i: (0, 0)),
            pl.BlockSpec((H, O), lambda i: (0, 0)),
        ],
        out_specs=pl.BlockSpec((BM, O), lambda i: (i, 0)),
        out_shape=jax.ShapeDtypeStruct((B, O), jnp.float32),
    )(orig, agg, W2aT, W2bT, WoutT)


def kernel(nodes, node_features, neigh_idx, W1, W2, Wout):
    nodes = nodes.astype(jnp.int32)
    neigh_idx = neigh_idx.astype(jnp.int32)
    orig, agg = _sc_gather_mean(nodes, node_features, neigh_idx)
    return _tc_dense(orig, agg, W2, Wout)


# speedup vs baseline: 1.0005x; 1.0005x over previous
import functools

import jax
import jax.numpy as jnp
from jax import lax
from jax.experimental import pallas as pl
from jax.experimental.pallas import tpu as pltpu
from jax.experimental.pallas import tpu_sc as plsc

N, D, S, B, H, O = 100000, 128, 16, 2048, 128, 64
NC, NS = 2, 16
NW = NC * NS
BPW = B // NW
CHUNK = 16
NCHUNK = BPW // CHUNK
ROWS = CHUNK * S
LANES = 16


def _sc_gather_mean(nodes, node_features, neigh_idx):
    mesh = plsc.VectorSubcoreMesh(core_axis_name="c", subcore_axis_name="s")

    @functools.partial(
        pl.kernel,
        out_type=(
            jax.ShapeDtypeStruct((B, D), jnp.float32),
            jax.ShapeDtypeStruct((B, D), jnp.float32),
        ),
        mesh=mesh,
        scratch_types=[
            pltpu.VMEM((BPW,), jnp.int32),
            pltpu.VMEM((BPW,), jnp.int32),
            pltpu.VMEM((BPW, 8 * S), jnp.int32),
            pltpu.VMEM((BPW * S,), jnp.int32),
            pltpu.VMEM((BPW, D), jnp.float32),
            pltpu.VMEM((ROWS, D), jnp.float32),
            pltpu.VMEM((CHUNK, D), jnp.float32),
            pltpu.SemaphoreType.DMA,
            pltpu.SemaphoreType.DMA,
        ],
    )
    def k(nodes_hbm, feats_hbm, nidx8_hbm, orig_out, agg_out,
          seeds_v, idx8_v, idrows_v, flat_v, orig_v, gbuf, aggbuf, sem, sem2):
        wid = lax.axis_index("s") * NC + lax.axis_index("c")
        base = wid * BPW

        pltpu.sync_copy(nodes_hbm.at[pl.ds(base, BPW)], seeds_v)
        for g in range(BPW // LANES):
            sl = pl.ds(g * LANES, LANES)
            idx8_v[sl] = lax.shift_right_logical(seeds_v[sl], 3)

        pltpu.async_copy(nidx8_hbm.at[idx8_v], idrows_v, sem)
        pltpu.async_copy(feats_hbm.at[seeds_v], orig_v, sem2)

        pltpu.make_async_copy(nidx8_hbm.at[idx8_v], idrows_v, sem).wait()
        for g in range(BPW // LANES):
            svec = seeds_v[pl.ds(g * LANES, LANES)]
            offs = (svec & 7) * S
            for j in range(LANES):
                i = g * LANES + j
                flat_v[pl.ds(i * S, S)] = idrows_v[i, pl.ds(offs[j], S)]

        pltpu.make_async_copy(feats_hbm.at[seeds_v], orig_v, sem2).wait()
        pltpu.sync_copy(orig_v, orig_out.at[pl.ds(base, BPW)])

        inv = jnp.float32(1.0 / S)
        for c in range(NCHUNK):
            pltpu.async_copy(
                feats_hbm.at[flat_v.at[pl.ds(c * ROWS, ROWS)]], gbuf, sem
            ).wait()

            def seed_body(s, carry):
                r0 = s * S
                for kk in range(D // LANES):
                    col = pl.ds(kk * LANES, LANES)
                    acc = gbuf[r0, col]
                    for j in range(1, S):
                        acc = acc + gbuf[r0 + j, col]
                    aggbuf[s, col] = acc * inv
                return carry

            lax.fori_loop(0, CHUNK, seed_body, 0)
            pltpu.sync_copy(aggbuf, agg_out.at[pl.ds(base + c * CHUNK, CHUNK)])

    return k(nodes, node_features, neigh_idx)


def _tc_dense(orig, agg, W2, Wout):
    W2aT = W2[:, :D].T
    W2bT = W2[:, D:].T
    WoutT = Wout.T
    BM = 256

    def body(o_ref, a_ref, w2a_ref, w2b_ref, wout_ref, out_ref):
        h = jnp.dot(o_ref[...], w2a_ref[...], preferred_element_type=jnp.float32)
        h = h + jnp.dot(a_ref[...], w2b_ref[...], preferred_element_type=jnp.float32)
        h = jnp.maximum(h, 0.0)
        logits = jnp.dot(h, wout_ref[...], preferred_element_type=jnp.float32)
        m = jnp.max(logits, axis=-1, keepdims=True)
        e = jnp.exp(logits - m)
        out_ref[...] = e / jnp.sum(e, axis=-1, keepdims=True)

    return pl.pallas_call(
        body,
        grid=(B // BM,),
        in_specs=[
            pl.BlockSpec((BM, D), lambda i: (i, 0)),
            pl.BlockSpec((BM, D), lambda i: (i, 0)),
            pl.BlockSpec((D, H), lambda i: (0, 0)),
            pl.BlockSpec((D, H), lambda i: (0, 0)),
            pl.BlockSpec((H, O), lambda i: (0, 0)),
        ],
        out_specs=pl.BlockSpec((BM, O), lambda i: (i, 0)),
        out_shape=jax.ShapeDtypeStruct((B, O), jnp.float32),
    )(orig, agg, W2aT, W2bT, WoutT)


def kernel(nodes, node_features, neigh_idx, W1, W2, Wout):
    nodes = nodes.astype(jnp.int32)
    neigh_idx8 = neigh_idx.astype(jnp.int32).reshape(N // 8, 8 * S)
    orig, agg = _sc_gather_mean(nodes, node_features, neigh_idx8)
    return _tc_dense(orig, agg, W2, Wout)


# speedup vs baseline: 1.9999x; 1.9989x over previous
import functools

import jax
import jax.numpy as jnp
from jax import lax
from jax.experimental import pallas as pl
from jax.experimental.pallas import tpu as pltpu
from jax.experimental.pallas import tpu_sc as plsc

N, D, S, B, H, O = 100000, 128, 16, 2048, 128, 64
NC, NS = 2, 16
NW = NC * NS
BPW = B // NW
LANES = 16
NIDS = BPW * S
FCH = 4
FROWS = NIDS // FCH


def _sc_gather_mean(nodes, node_features, nidx_sm):
    mesh = plsc.VectorSubcoreMesh(core_axis_name="c", subcore_axis_name="s")

    @functools.partial(
        pl.kernel,
        out_type=(
            jax.ShapeDtypeStruct((B, D), jnp.float32),
            jax.ShapeDtypeStruct((B, D), jnp.float32),
        ),
        mesh=mesh,
        compiler_params=pltpu.CompilerParams(use_tc_tiling_on_sc=False),
        scratch_types=[
            pltpu.VMEM((BPW,), jnp.int32),
            pltpu.VMEM((NIDS,), jnp.int32),
            pltpu.VMEM((NIDS,), jnp.int32),
            pltpu.VMEM((BPW, D), jnp.float32),
            pltpu.VMEM((FROWS, D), jnp.float32),
            pltpu.VMEM((FROWS, D), jnp.float32),
            pltpu.VMEM((BPW, D), jnp.float32),
            pltpu.SemaphoreType.DMA,
            pltpu.SemaphoreType.DMA,
            pltpu.SemaphoreType.DMA,
            pltpu.SemaphoreType.DMA,
        ],
    )
    def k(nodes_hbm, feats_hbm, nidx_hbm, orig_out, agg_out,
          seeds_v, cidx_v, flat_v, orig_v, gbuf0, gbuf1, acc_v,
          semi, semo, semf0, semf1):
        wid = lax.axis_index("s") * NC + lax.axis_index("c")
        base = wid * BPW

        pltpu.sync_copy(nodes_hbm.at[pl.ds(base, BPW)], seeds_v)

        for g in range(BPW // LANES):
            nvec = seeds_v[pl.ds(g * LANES, LANES)]
            for j in range(S):
                cidx_v[pl.ds(j * BPW + g * LANES, LANES)] = nvec + (j * N)

        idgather = pltpu.async_copy(nidx_hbm.at[cidx_v], flat_v, semi)
        origather = pltpu.async_copy(feats_hbm.at[seeds_v], orig_v, semo)
        idgather.wait()

        bufs = (gbuf0, gbuf1)
        sems = (semf0, semf1)

        def fire(c):
            pltpu.async_copy(
                feats_hbm.at[flat_v.at[pl.ds(c * FROWS, FROWS)]],
                bufs[c % 2], sems[c % 2],
            )

        fire(0)
        fire(1)
        inv = jnp.float32(1.0 / S)
        for c in range(FCH):
            buf = bufs[c % 2]
            pltpu.make_async_copy(
                feats_hbm.at[flat_v.at[pl.ds(c * FROWS, FROWS)]],
                buf, sems[c % 2],
            ).wait()

            if c == 0:
                def acc0_body(si, carry):
                    for kk in range(D // LANES):
                        col = pl.ds(kk * LANES, LANES)
                        v = (buf[si, col] + buf[BPW + si, col]
                             + buf[2 * BPW + si, col] + buf[3 * BPW + si, col])
                        acc_v[si, col] = v
                    return carry
                lax.fori_loop(0, BPW, acc0_body, 0)
            else:
                def accn_body(si, carry):
                    for kk in range(D // LANES):
                        col = pl.ds(kk * LANES, LANES)
                        v = (buf[si, col] + buf[BPW + si, col]
                             + buf[2 * BPW + si, col] + buf[3 * BPW + si, col])
                        acc_v[si, col] = acc_v[si, col] + v
                    return carry
                lax.fori_loop(0, BPW, accn_body, 0)

            if c + 2 < FCH:
                fire(c + 2)

        origather.wait()
        pltpu.sync_copy(orig_v, orig_out.at[pl.ds(base, BPW)])

        def scale_body(si, carry):
            for kk in range(D // LANES):
                col = pl.ds(kk * LANES, LANES)
                acc_v[si, col] = acc_v[si, col] * inv
            return carry

        lax.fori_loop(0, BPW, scale_body, 0)
        pltpu.sync_copy(acc_v, agg_out.at[pl.ds(base, BPW)])

    return k(nodes, node_features, nidx_sm)


def _tc_dense(orig, agg, W2, Wout):
    W2aT = W2[:, :D].T
    W2bT = W2[:, D:].T
    WoutT = Wout.T
    BM = 256

    def body(o_ref, a_ref, w2a_ref, w2b_ref, wout_ref, out_ref):
        h = jnp.dot(o_ref[...], w2a_ref[...], preferred_element_type=jnp.float32)
        h = h + jnp.dot(a_ref[...], w2b_ref[...], preferred_element_type=jnp.float32)
        h = jnp.maximum(h, 0.0)
        logits = jnp.dot(h, wout_ref[...], preferred_element_type=jnp.float32)
        m = jnp.max(logits, axis=-1, keepdims=True)
        e = jnp.exp(logits - m)
        out_ref[...] = e / jnp.sum(e, axis=-1, keepdims=True)

    return pl.pallas_call(
        body,
        grid=(B // BM,),
        in_specs=[
            pl.BlockSpec((BM, D), lambda i: (i, 0)),
            pl.BlockSpec((BM, D), lambda i: (i, 0)),
            pl.BlockSpec((D, H), lambda i: (0, 0)),
            pl.BlockSpec((D, H), lambda i: (0, 0)),
            pl.BlockSpec((H, O), lambda i: (0, 0)),
        ],
        out_specs=pl.BlockSpec((BM, O), lambda i: (i, 0)),
        out_shape=jax.ShapeDtypeStruct((B, O), jnp.float32),
    )(orig, agg, W2aT, W2bT, WoutT)


def kernel(nodes, node_features, neigh_idx, W1, W2, Wout):
    nodes = nodes.astype(jnp.int32)
    nidx_sm = jnp.transpose(neigh_idx.astype(jnp.int32)).reshape(N * S)
    orig, agg = _sc_gather_mean(nodes, node_features, nidx_sm)
    return _tc_dense(orig, agg, W2, Wout)


# speedup vs baseline: 2.1707x; 1.0854x over previous
import functools

import jax
import jax.numpy as jnp
from jax import lax
from jax.experimental import pallas as pl
from jax.experimental.pallas import tpu as pltpu
from jax.experimental.pallas import tpu_sc as plsc

N, D, S, B, H, O = 100000, 128, 16, 2048, 128, 64
NC, NS = 2, 16
NW = NC * NS
BPW = B // NW
LANES = 16
NIDS = BPW * S
FCH = 4
FROWS = NIDS // FCH


def _sc_gather_mean(nodes, node_features, nidx_sm):
    mesh = plsc.VectorSubcoreMesh(core_axis_name="c", subcore_axis_name="s")

    @functools.partial(
        pl.kernel,
        out_type=(
            jax.ShapeDtypeStruct((B, D), jnp.float32),
            jax.ShapeDtypeStruct((B, D), jnp.float32),
        ),
        mesh=mesh,
        compiler_params=pltpu.CompilerParams(use_tc_tiling_on_sc=False),
        scratch_types=[
            pltpu.VMEM((BPW,), jnp.int32),
            pltpu.VMEM((NIDS,), jnp.int32),
            pltpu.VMEM((NIDS,), jnp.int32),
            pltpu.VMEM((BPW, D), jnp.float32),
            pltpu.VMEM((FROWS, D), jnp.float32),
            pltpu.VMEM((FROWS, D), jnp.float32),
            pltpu.VMEM((BPW, D), jnp.float32),
            pltpu.SemaphoreType.DMA,
            pltpu.SemaphoreType.DMA,
            pltpu.SemaphoreType.DMA,
            pltpu.SemaphoreType.DMA,
        ],
    )
    def k(nodes_hbm, feats_hbm, nidx_hbm, orig_out, agg_out,
          seeds_v, cidx_v, flat_v, orig_v, gbuf0, gbuf1, acc_v,
          semi, semo, semf0, semf1):
        wid = lax.axis_index("s") * NC + lax.axis_index("c")
        base = wid * BPW

        pltpu.sync_copy(nodes_hbm.at[pl.ds(base, BPW)], seeds_v)

        for g in range(BPW // LANES):
            nvec = seeds_v[pl.ds(g * LANES, LANES)]
            for j in range(S):
                cidx_v[pl.ds(j * BPW + g * LANES, LANES)] = nvec + (j * N)

        idgather = pltpu.async_copy(nidx_hbm.at[cidx_v], flat_v, semi)
        origather = pltpu.async_copy(feats_hbm.at[seeds_v], orig_v, semo)
        idgather.wait()

        bufs = (gbuf0, gbuf1)
        sems = (semf0, semf1)

        def fire(c):
            pltpu.async_copy(
                feats_hbm.at[flat_v.at[pl.ds(c * FROWS, FROWS)]],
                bufs[c % 2], sems[c % 2],
            )

        fire(0)
        fire(1)
        inv = jnp.float32(1.0 / S)
        for c in range(FCH):
            buf = bufs[c % 2]
            pltpu.make_async_copy(
                feats_hbm.at[flat_v.at[pl.ds(c * FROWS, FROWS)]],
                buf, sems[c % 2],
            ).wait()

            if c == 0:
                def acc0_body(si, carry):
                    for kk in range(D // LANES):
                        col = pl.ds(kk * LANES, LANES)
                        v = (buf[si, col] + buf[BPW + si, col]
                             + buf[2 * BPW + si, col] + buf[3 * BPW + si, col])
                        acc_v[si, col] = v
                    return carry
                lax.fori_loop(0, BPW, acc0_body, 0)
            else:
                def accn_body(si, carry):
                    for kk in range(D // LANES):
                        col = pl.ds(kk * LANES, LANES)
                        v = (buf[si, col] + buf[BPW + si, col]
                             + buf[2 * BPW + si, col] + buf[3 * BPW + si, col])
                        acc_v[si, col] = acc_v[si, col] + v
                    return carry
                lax.fori_loop(0, BPW, accn_body, 0)

            if c + 2 < FCH:
                fire(c + 2)

        origather.wait()
        pltpu.sync_copy(orig_v, orig_out.at[pl.ds(base, BPW)])

        def scale_body(si, carry):
            for kk in range(D // LANES):
                col = pl.ds(kk * LANES, LANES)
                acc_v[si, col] = acc_v[si, col] * inv
            return carry

        lax.fori_loop(0, BPW, scale_body, 0)
        pltpu.sync_copy(acc_v, agg_out.at[pl.ds(base, BPW)])

    return k(nodes, node_features, nidx_sm)


def _tc_dense(orig, agg, W2, Wout):
    BM = 1024
    dn = (((1,), (1,)), ((), ()))

    def body(o_ref, a_ref, w2_ref, wout_ref, out_ref):
        h = lax.dot_general(o_ref[...], w2_ref[:, :D], dn,
                            preferred_element_type=jnp.float32)
        h = h + lax.dot_general(a_ref[...], w2_ref[:, D:], dn,
                                preferred_element_type=jnp.float32)
        h = jnp.maximum(h, 0.0)
        logits = lax.dot_general(h, wout_ref[...], dn,
                                 preferred_element_type=jnp.float32)
        m = jnp.max(logits, axis=-1, keepdims=True)
        e = jnp.exp(logits - m)
        out_ref[...] = e / jnp.sum(e, axis=-1, keepdims=True)

    return pl.pallas_call(
        body,
        grid=(B // BM,),
        in_specs=[
            pl.BlockSpec((BM, D), lambda i: (i, 0)),
            pl.BlockSpec((BM, D), lambda i: (i, 0)),
            pl.BlockSpec((H, 2 * D), lambda i: (0, 0)),
            pl.BlockSpec((O, H), lambda i: (0, 0)),
        ],
        out_specs=pl.BlockSpec((BM, O), lambda i: (i, 0)),
        out_shape=jax.ShapeDtypeStruct((B, O), jnp.float32),
    )(orig, agg, W2, Wout)


def kernel(nodes, node_features, neigh_idx, W1, W2, Wout):
    nodes = nodes.astype(jnp.int32)
    nidx_sm = jnp.transpose(neigh_idx.astype(jnp.int32)).reshape(N * S)
    orig, agg = _sc_gather_mean(nodes, node_features, nidx_sm)
    return _tc_dense(orig, agg, W2, Wout)
